# baseline (speedup 1.0000x reference)
import jax
import jax.numpy as jnp
from jax.experimental import pallas as pl

_TS = 256


def _add_kernel(x_ref, pos_ref, o_ref):
    o_ref[...] = x_ref[...] + pos_ref[...]


def kernel(x, pos_table):
    B, T, E = x.shape
    return pl.pallas_call(
        _add_kernel,
        grid=(T // _TS,),
        in_specs=[
            pl.BlockSpec((B, _TS, E), lambda t: (0, t, 0)),
            pl.BlockSpec((_TS, E), lambda t: (t, 0)),
        ],
        out_specs=pl.BlockSpec((B, _TS, E), lambda t: (0, t, 0)),
        out_shape=jax.ShapeDtypeStruct((B, T, E), x.dtype),
    )(x, pos_table)
